# baseline (speedup 1.0000x reference)
import functools
import numpy as np
import jax
import jax.numpy as jnp
from jax import lax
from jax.experimental import pallas as pl
from jax.experimental.pallas import tpu as pltpu
from jax.experimental.pallas import tpu_sc as plsc

_B, _C, _F, _T = 16, 1, 128, 4096
_L = 16
_NC, _NS = 2, 16
_NW = _NC * _NS
_ROWS = _B * _F
_RPW = _ROWS // _NW
_CH = 4
_NCHUNK = _RPW // _CH
_TG = _T // _L

_factors_f64 = np.linspace(1.0, 3.0, _B)
_ti = np.arange(_T)[None, :] / _factors_f64[:, None]
_floor_np = np.floor(_ti).astype(np.int32)
_frac_np = (_ti - np.floor(_ti)).astype(np.float32)
_factors_np = _factors_f64.astype(np.float32)

_mesh = plsc.VectorSubcoreMesh(core_axis_name="c", subcore_axis_name="s")


@functools.partial(
    pl.kernel,
    out_type=(
        jax.ShapeDtypeStruct((_ROWS * _T,), jnp.float32),
        jax.ShapeDtypeStruct((_B,), jnp.int32),
    ),
    mesh=_mesh,
    compiler_params=pltpu.CompilerParams(needs_layout_passes=False),
    scratch_types=[
        pltpu.VMEM((_T,), jnp.int32),
        pltpu.VMEM((_T,), jnp.float32),
        pltpu.VMEM((_CH * _T,), jnp.float32),
        pltpu.VMEM((_CH * _T,), jnp.float32),
        pltpu.VMEM((_B,), jnp.int32),
        pltpu.VMEM((_B,), jnp.float32),
        pltpu.VMEM((_B,), jnp.int32),
    ],
)
def _warp_kernel(x_hbm, seqlen_hbm, floor_hbm, frac_hbm, fac_hbm,
                 out_hbm, nsl_hbm,
                 floor_v, frac_v, xin_v, xout_v, seq_v, fac_v, nsl_v):
    wid = lax.axis_index("s") * _NC + lax.axis_index("c")
    b = wid // 2
    row0 = b * _F + (wid % 2) * _RPW

    @pl.when(wid == 0)
    def _():
        pltpu.sync_copy(seqlen_hbm, seq_v)
        pltpu.sync_copy(fac_hbm, fac_v)
        s = seq_v[...].astype(jnp.float32) * fac_v[...]
        nsl_v[...] = jnp.minimum(s, jnp.float32(_T)).astype(jnp.int32)
        pltpu.sync_copy(nsl_v, nsl_hbm)

    pltpu.sync_copy(floor_hbm.at[b], floor_v)
    pltpu.sync_copy(frac_hbm.at[b], frac_v)

    for ch in range(_NCHUNK):
        r0 = row0 + ch * _CH
        pltpu.sync_copy(x_hbm.at[pl.ds(r0 * _T, _CH * _T)], xin_v)

        def tbody(i, _):
            off = i * _L
            fi = floor_v[pl.ds(off, _L)]
            fr = frac_v[pl.ds(off, _L)]
            fi1 = jnp.minimum(fi + 1, _T - 1)
            for r in range(_CH):
                a = plsc.load_gather(xin_v, [fi + (r * _T)])
                c = plsc.load_gather(xin_v, [fi1 + (r * _T)])
                xout_v[pl.ds(r * _T + off, _L)] = a + fr * (c - a)
            return 0

        lax.fori_loop(0, _TG, tbody, 0)
        pltpu.sync_copy(xout_v, out_hbm.at[pl.ds(r0 * _T, _CH * _T)])


def kernel(x, seq_len):
    xf = x.reshape(_ROWS * _T)
    out_flat, new_seq_len = _warp_kernel(
        xf, seq_len,
        jnp.asarray(_floor_np), jnp.asarray(_frac_np), jnp.asarray(_factors_np),
    )
    return out_flat.reshape(_B, _C, _F, _T), new_seq_len


# speedup vs baseline: 2.2859x; 2.2859x over previous
import functools
import numpy as np
import jax
import jax.numpy as jnp
from jax import lax
from jax.experimental import pallas as pl
from jax.experimental.pallas import tpu as pltpu
from jax.experimental.pallas import tpu_sc as plsc

_B, _C, _F, _T = 16, 1, 128, 4096
_L = 16
_NC, _NS = 2, 16
_NW = _NC * _NS
_ROWS = _B * _F
_RPW = _ROWS // _NW
_CH = 4
_NCHUNK = _RPW // _CH
_TG = _T // _L

_factors_f64 = np.linspace(1.0, 3.0, _B)
_ti = np.arange(_T)[None, :] / _factors_f64[:, None]
_floor_np = np.floor(_ti).astype(np.int32)
_frac_np = (_ti - np.floor(_ti)).astype(np.float32)
_factors_np = _factors_f64.astype(np.float32)

_mesh = plsc.VectorSubcoreMesh(core_axis_name="c", subcore_axis_name="s")


@functools.partial(
    pl.kernel,
    out_type=(
        jax.ShapeDtypeStruct((_ROWS * _T,), jnp.float32),
        jax.ShapeDtypeStruct((_B,), jnp.int32),
    ),
    mesh=_mesh,
    compiler_params=pltpu.CompilerParams(needs_layout_passes=False),
    scratch_types=[
        pltpu.VMEM((_T,), jnp.int32),
        pltpu.VMEM((_T,), jnp.float32),
        pltpu.VMEM((_CH * _T,), jnp.float32),
        pltpu.VMEM((_CH * _T,), jnp.float32),
        pltpu.VMEM((_CH * _T,), jnp.float32),
        pltpu.VMEM((_CH * _T,), jnp.float32),
        pltpu.VMEM((_B,), jnp.int32),
        pltpu.VMEM((_B,), jnp.float32),
        pltpu.VMEM((_B,), jnp.int32),
        pltpu.SemaphoreType.DMA,
        pltpu.SemaphoreType.DMA,
        pltpu.SemaphoreType.DMA,
        pltpu.SemaphoreType.DMA,
    ],
)
def _warp_kernel(x_hbm, seqlen_hbm, floor_hbm, frac_hbm, fac_hbm,
                 out_hbm, nsl_hbm,
                 floor_v, frac_v, xin0, xin1, xout0, xout1,
                 seq_v, fac_v, nsl_v,
                 isem0, isem1, osem0, osem1):
    wid = lax.axis_index("s") * _NC + lax.axis_index("c")
    b = wid // 2
    row0 = b * _F + (wid % 2) * _RPW
    xin = (xin0, xin1)
    xout = (xout0, xout1)
    isem = (isem0, isem1)
    osem = (osem0, osem1)

    @pl.when(wid == 0)
    def _():
        pltpu.sync_copy(seqlen_hbm, seq_v)
        pltpu.sync_copy(fac_hbm, fac_v)
        s = seq_v[...].astype(jnp.float32) * fac_v[...]
        nsl_v[...] = jnp.minimum(s, jnp.float32(_T)).astype(jnp.int32)
        pltpu.sync_copy(nsl_v, nsl_hbm)

    pltpu.sync_copy(floor_hbm.at[b], floor_v)
    pltpu.sync_copy(frac_hbm.at[b], frac_v)

    def in_copy(ch):
        r0 = row0 + ch * _CH
        return pltpu.async_copy(
            x_hbm.at[pl.ds(r0 * _T, _CH * _T)], xin[ch % 2], isem[ch % 2])

    def out_copy(ch):
        r0 = row0 + ch * _CH
        return pltpu.async_copy(
            xout[ch % 2], out_hbm.at[pl.ds(r0 * _T, _CH * _T)], osem[ch % 2])

    hin = {0: in_copy(0)}
    hout = {}
    for ch in range(_NCHUNK):
        if ch + 1 < _NCHUNK:
            hin[ch + 1] = in_copy(ch + 1)
        hin[ch].wait()
        if ch >= 2:
            hout[ch - 2].wait()
        src = xin[ch % 2]
        dst = xout[ch % 2]

        @plsc.parallel_loop(0, _TG, unroll=4)
        def _(i):
            off = i * _L
            fi = floor_v[pl.ds(off, _L)]
            fr = frac_v[pl.ds(off, _L)]
            fi1 = jnp.minimum(fi + 1, _T - 1)
            for r in range(_CH):
                a = plsc.load_gather(src, [fi + (r * _T)])
                c = plsc.load_gather(src, [fi1 + (r * _T)])
                dst[pl.ds(r * _T + off, _L)] = a + fr * (c - a)

        hout[ch] = out_copy(ch)
    hout[_NCHUNK - 2].wait()
    hout[_NCHUNK - 1].wait()


def kernel(x, seq_len):
    xf = x.reshape(_ROWS * _T)
    out_flat, new_seq_len = _warp_kernel(
        xf, seq_len,
        jnp.asarray(_floor_np), jnp.asarray(_frac_np), jnp.asarray(_factors_np),
    )
    return out_flat.reshape(_B, _C, _F, _T), new_seq_len


# speedup vs baseline: 4.5730x; 2.0006x over previous
import functools
import numpy as np
import jax
import jax.numpy as jnp
from jax import lax
from jax.experimental import pallas as pl
from jax.experimental.pallas import tpu as pltpu
from jax.experimental.pallas import tpu_sc as plsc

_B, _C, _F, _T = 16, 1, 128, 4096
_L = 16
_NC, _NS = 2, 16
_NW = _NC * _NS
_ROWS = _B * _F
_RPW = _ROWS // _NW
_CH = 4
_NCHUNK = _RPW // _CH
_TG = _T // _L
_TT = _T // 128
_TR = _ROWS // 8

_factors_f64 = np.linspace(1.0, 3.0, _B)
_ti = np.arange(_T)[None, :] / _factors_f64[:, None]
_floor_np = np.floor(_ti).astype(np.int32).reshape(-1)
_frac_np = (_ti - np.floor(_ti)).astype(np.float32).reshape(-1)
_factors_np = _factors_f64.astype(np.float32)

_mesh = plsc.VectorSubcoreMesh(core_axis_name="c", subcore_axis_name="s")


@functools.partial(
    pl.kernel,
    out_type=(
        jax.ShapeDtypeStruct((_TR, _TT, 8, 128), jnp.float32),
        jax.ShapeDtypeStruct((_B,), jnp.int32),
    ),
    mesh=_mesh,
    compiler_params=pltpu.CompilerParams(needs_layout_passes=False),
    scratch_types=[
        pltpu.VMEM((_T,), jnp.int32),
        pltpu.VMEM((_T,), jnp.float32),
        pltpu.VMEM((_CH, _TT, 128), jnp.float32),
        pltpu.VMEM((_CH, _TT, 128), jnp.float32),
        pltpu.VMEM((_CH, _TT, 128), jnp.float32),
        pltpu.VMEM((_CH, _TT, 128), jnp.float32),
        pltpu.VMEM((_B,), jnp.int32),
        pltpu.VMEM((_B,), jnp.float32),
        pltpu.VMEM((_B,), jnp.int32),
        pltpu.SemaphoreType.DMA,
        pltpu.SemaphoreType.DMA,
        pltpu.SemaphoreType.DMA,
        pltpu.SemaphoreType.DMA,
    ],
)
def _warp_kernel(x_hbm, seqlen_hbm, floor_hbm, frac_hbm, fac_hbm,
                 out_hbm, nsl_hbm,
                 floor_v, frac_v, xin0, xin1, xout0, xout1,
                 seq_v, fac_v, nsl_v,
                 isem0, isem1, osem0, osem1):
    wid = lax.axis_index("s") * _NC + lax.axis_index("c")
    b = wid // 2
    row0 = b * _F + (wid % 2) * _RPW
    xin = (xin0, xin1)
    xout = (xout0, xout1)
    isem = (isem0, isem1)
    osem = (osem0, osem1)

    @pl.when(wid == 0)
    def _():
        pltpu.sync_copy(seqlen_hbm, seq_v)
        pltpu.sync_copy(fac_hbm, fac_v)
        s = seq_v[...].astype(jnp.float32) * fac_v[...]
        nsl_v[...] = jnp.minimum(s, jnp.float32(_T)).astype(jnp.int32)
        pltpu.sync_copy(nsl_v, nsl_hbm)

    pltpu.sync_copy(floor_hbm.at[pl.ds(b * _T, _T)], floor_v)
    pltpu.sync_copy(frac_hbm.at[pl.ds(b * _T, _T)], frac_v)

    def in_copy(ch):
        r0 = row0 + ch * _CH
        hs = []
        for r in range(_CH):
            row = r0 + r
            hs.append(pltpu.async_copy(
                x_hbm.at[row // 8, :, row % 8, :],
                xin[ch % 2].at[r], isem[ch % 2]))
        return hs

    def out_copy(ch):
        r0 = row0 + ch * _CH
        hs = []
        for r in range(_CH):
            row = r0 + r
            hs.append(pltpu.async_copy(
                xout[ch % 2].at[r],
                out_hbm.at[row // 8, :, row % 8, :], osem[ch % 2]))
        return hs

    hin = {0: in_copy(0)}
    hout = {}
    for ch in range(_NCHUNK):
        if ch + 1 < _NCHUNK:
            hin[ch + 1] = in_copy(ch + 1)
        for h in hin.pop(ch):
            h.wait()
        if ch >= 2:
            for h in hout.pop(ch - 2):
                h.wait()
        src = xin[ch % 2]
        dst = xout[ch % 2]

        @plsc.parallel_loop(0, _TG, unroll=4)
        def _(i):
            off = i * _L
            fi = floor_v[pl.ds(off, _L)]
            fr = frac_v[pl.ds(off, _L)]
            fi1 = jnp.minimum(fi + 1, _T - 1)
            fi_hi = lax.shift_right_logical(fi, 7)
            fi_lo = lax.bitwise_and(fi, 127)
            fi1_hi = lax.shift_right_logical(fi1, 7)
            fi1_lo = lax.bitwise_and(fi1, 127)
            tc = i // 8
            lo = (i % 8) * _L
            for r in range(_CH):
                ridx = jnp.full((_L,), r, jnp.int32)
                a = plsc.load_gather(src, [ridx, fi_hi, fi_lo])
                c = plsc.load_gather(src, [ridx, fi1_hi, fi1_lo])
                dst[r, tc, pl.ds(lo, _L)] = a + fr * (c - a)

        hout[ch] = out_copy(ch)
    for h in hout.pop(_NCHUNK - 2):
        h.wait()
    for h in hout.pop(_NCHUNK - 1):
        h.wait()


def kernel(x, seq_len):
    xt = x.reshape(_TR, 8, _TT, 128).swapaxes(1, 2)
    out_t, new_seq_len = _warp_kernel(
        xt, seq_len,
        jnp.asarray(_floor_np), jnp.asarray(_frac_np), jnp.asarray(_factors_np),
    )
    out = out_t.swapaxes(1, 2).reshape(_B, _C, _F, _T)
    return out, new_seq_len
